# Initial kernel scaffold; baseline (speedup 1.0000x reference)
import functools

import jax
import jax.numpy as jnp
from jax import lax
from jax.experimental import pallas as pl
from jax.experimental.pallas import tpu as pltpu
from jax.experimental.pallas import tpu_sc as plsc

N = 10000
E = 320000
D_IN = 128
D_H = 128
D_OUT = 64

NC = 2
NS = 16
NW = NC * NS
CHUNK = 128
IDX_ROWS = (E + CHUNK - 1) // CHUNK
IDX_ROWS_PAD = ((IDX_ROWS + NW - 1) // NW) * NW
E_PAD = IDX_ROWS_PAD * CHUNK
R_PER_W = IDX_ROWS_PAD // NW
N_PAD = 10240
DEG_W = 16
ROWS_PER_TILE = N_PAD // NS


def _fill(ref, val, ncols):
    def body(i, carry):
        for k in range(ncols // 16):
            ref[i, pl.ds(16 * k, 16)] = jnp.full((16,), val, jnp.float32)
        return carry
    lax.fori_loop(0, CHUNK, body, 0)


def _make_seg_sum(d, with_deg):
    out_type = [jax.ShapeDtypeStruct((NC, N_PAD, d), jnp.float32)]
    if with_deg:
        out_type.append(jax.ShapeDtypeStruct((NC, N_PAD, DEG_W), jnp.float32))

    scratch = [
        pltpu.VMEM((R_PER_W, CHUNK), jnp.int32),
        pltpu.VMEM((R_PER_W, CHUNK), jnp.int32),
        pltpu.VMEM((CHUNK, d), jnp.float32),
        pltpu.VMEM_SHARED((N_PAD, d), jnp.float32),
        pltpu.SemaphoreType.DMA,
    ]
    if with_deg:
        scratch.append(pltpu.VMEM((CHUNK, DEG_W), jnp.float32))
        scratch.append(pltpu.VMEM_SHARED((N_PAD, DEG_W), jnp.float32))

    mesh = plsc.VectorSubcoreMesh(core_axis_name="c", subcore_axis_name="s")

    def body(x_hbm, src_hbm, dst_hbm, *refs):
        if with_deg:
            out_hbm, deg_hbm, srcv, dstv, rows, acc_sh, sem, ones, deg_sh = refs
        else:
            out_hbm, srcv, dstv, rows, acc_sh, sem = refs
        c = lax.axis_index("c")
        s = lax.axis_index("s")
        w = s * NC + c

        _fill(rows, 0.0, d)
        if with_deg:
            _fill(ones, 0.0, DEG_W)

        def zero_body(k, carry):
            r0 = s * ROWS_PER_TILE + k * CHUNK
            pltpu.sync_copy(rows, acc_sh.at[pl.ds(r0, CHUNK)])
            if with_deg:
                pltpu.sync_copy(ones, deg_sh.at[pl.ds(r0, CHUNK)])
            return carry
        lax.fori_loop(0, ROWS_PER_TILE // CHUNK, zero_body, 0)

        if with_deg:
            _fill(ones, 1.0, DEG_W)
        plsc.subcore_barrier()

        pltpu.sync_copy(src_hbm.at[pl.ds(w * R_PER_W, R_PER_W)], srcv)
        pltpu.sync_copy(dst_hbm.at[pl.ds(w * R_PER_W, R_PER_W)], dstv)

        def edge_body(j, carry):
            pltpu.async_copy(x_hbm.at[srcv.at[j]], rows, sem).wait()
            pltpu.sync_copy(rows, acc_sh.at[dstv.at[j]], add=True)
            if with_deg:
                pltpu.sync_copy(ones, deg_sh.at[dstv.at[j]], add=True)
            return carry
        lax.fori_loop(0, R_PER_W, edge_body, 0)
        plsc.subcore_barrier()

        def out_body(k, carry):
            r0 = s * ROWS_PER_TILE + k * CHUNK
            pltpu.sync_copy(acc_sh.at[pl.ds(r0, CHUNK)], rows)
            pltpu.sync_copy(rows, out_hbm.at[c, pl.ds(r0, CHUNK)])
            if with_deg:
                pltpu.sync_copy(deg_sh.at[pl.ds(r0, CHUNK)], ones)
                pltpu.sync_copy(ones, deg_hbm.at[c, pl.ds(r0, CHUNK)])
            return carry
        lax.fori_loop(0, ROWS_PER_TILE // CHUNK, out_body, 0)

    return pl.kernel(body, out_type=out_type, mesh=mesh, scratch_types=scratch)


_seg_sum_deg = _make_seg_sum(D_IN, with_deg=True)
_seg_sum_64 = _make_seg_sum(D_OUT, with_deg=False)

ROW_BLK = 2000
GRID = N // ROW_BLK


def _tc1_body(x_ref, agg_ref, deg_ref, w1t_ref, b1_ref, w2t_ref, p_ref):
    agg = agg_ref[0] + agg_ref[1] + x_ref[...]
    deg = deg_ref[0, :, 0:1] + deg_ref[1, :, 0:1]
    hn = agg * (1.0 / (deg + 1.0))
    h = jnp.dot(hn, w1t_ref[...], preferred_element_type=jnp.float32)
    h = h + b1_ref[...]
    h = jnp.where(h >= 0.0, h, 0.01 * h)
    p_ref[...] = jnp.dot(h, w2t_ref[...], preferred_element_type=jnp.float32)


def _tc2_body(p_ref, agg_ref, deg_ref, b2_ref, o_ref):
    agg = agg_ref[0] + agg_ref[1] + p_ref[...]
    deg = deg_ref[0, :, 0:1] + deg_ref[1, :, 0:1]
    o_ref[...] = agg * (1.0 / (deg + 1.0)) + b2_ref[...]


_tc1 = pl.pallas_call(
    _tc1_body,
    grid=(GRID,),
    in_specs=[
        pl.BlockSpec((ROW_BLK, D_IN), lambda i: (i, 0)),
        pl.BlockSpec((NC, ROW_BLK, D_IN), lambda i: (0, i, 0)),
        pl.BlockSpec((NC, ROW_BLK, DEG_W), lambda i: (0, i, 0)),
        pl.BlockSpec((D_IN, D_H), lambda i: (0, 0)),
        pl.BlockSpec((1, D_H), lambda i: (0, 0)),
        pl.BlockSpec((D_H, D_OUT), lambda i: (0, 0)),
    ],
    out_specs=pl.BlockSpec((ROW_BLK, D_OUT), lambda i: (i, 0)),
    out_shape=jax.ShapeDtypeStruct((N, D_OUT), jnp.float32),
)

_tc2 = pl.pallas_call(
    _tc2_body,
    grid=(GRID,),
    in_specs=[
        pl.BlockSpec((ROW_BLK, D_OUT), lambda i: (i, 0)),
        pl.BlockSpec((NC, ROW_BLK, D_OUT), lambda i: (0, i, 0)),
        pl.BlockSpec((NC, ROW_BLK, DEG_W), lambda i: (0, i, 0)),
        pl.BlockSpec((1, D_OUT), lambda i: (0, 0)),
    ],
    out_specs=pl.BlockSpec((ROW_BLK, D_OUT), lambda i: (i, 0)),
    out_shape=jax.ShapeDtypeStruct((N, D_OUT), jnp.float32),
)


def kernel(in_feat, edge_index, W1, b1, W2, b2):
    src = edge_index[0]
    dst = edge_index[1]
    pad = E_PAD - E
    srcp = jnp.concatenate([src, jnp.zeros((pad,), jnp.int32)])
    srcp = srcp.reshape(IDX_ROWS_PAD, CHUNK)
    dstp = jnp.concatenate([dst, jnp.full((pad,), N, jnp.int32)])
    dstp = dstp.reshape(IDX_ROWS_PAD, CHUNK)

    agg1p, degp = _seg_sum_deg(in_feat, srcp, dstp)
    p = _tc1(in_feat, agg1p, degp, W1.T, b1.reshape(1, D_H), W2.T)
    (agg2p,) = _seg_sum_64(p, srcp, dstp)
    return _tc2(p, agg2p, degp, b2.reshape(1, D_OUT))


# speedup vs baseline: 2.9292x; 2.9292x over previous
import jax
import jax.numpy as jnp
from jax import lax
from jax.experimental import pallas as pl
from jax.experimental.pallas import tpu as pltpu
from jax.experimental.pallas import tpu_sc as plsc

N = 10000
E = 320000
D_IN = 128
D_H = 128
D_OUT = 64

NC = 2
NS = 16
NW = NC * NS
CHUNK = 128
IDX_ROWS = (E + CHUNK - 1) // CHUNK
IDX_ROWS_PAD = ((IDX_ROWS + 8 * NW - 1) // (8 * NW)) * (8 * NW)
E_PAD = IDX_ROWS_PAD * CHUNK
R_PER_W = IDX_ROWS_PAD // NW
IDX_BLK = 16
N_PAD = 10240
ROWS_PER_TILE = N_PAD // NS


def _make_seg_sum(d, gather):
    out_type = [jax.ShapeDtypeStruct((NC * N_PAD, d), jnp.float32)]

    scratch = [
        pltpu.VMEM((IDX_BLK, CHUNK), jnp.int32),
        pltpu.VMEM((IDX_BLK, CHUNK), jnp.int32),
        pltpu.VMEM((CHUNK, d), jnp.float32),
        pltpu.VMEM_SHARED((N_PAD, d), jnp.float32),
        pltpu.SemaphoreType.DMA,
    ]

    def body(x_hbm, src_hbm, dst_hbm, z_hbm, out_hbm,
             srcv, dstv, rows, acc_sh, sem):
        c = lax.axis_index("c")
        s = lax.axis_index("s")
        w = s * NC + c

        pltpu.sync_copy(z_hbm, rows)

        def zero_body(k, carry):
            r0 = s * ROWS_PER_TILE + k * CHUNK
            pltpu.sync_copy(rows, acc_sh.at[pl.ds(r0, CHUNK)])
            return carry
        lax.fori_loop(0, ROWS_PER_TILE // CHUNK, zero_body, 0)
        plsc.subcore_barrier()

        if not gather:
            pltpu.sync_copy(x_hbm, rows)

        def blk_body(b, carry):
            r0 = w * R_PER_W + b * IDX_BLK
            if gather:
                pltpu.sync_copy(src_hbm.at[pl.ds(r0, IDX_BLK)], srcv)
            pltpu.sync_copy(dst_hbm.at[pl.ds(r0, IDX_BLK)], dstv)

            for j in range(IDX_BLK):
                if gather:
                    pltpu.async_copy(x_hbm.at[srcv.at[j]], rows, sem).wait()
                pltpu.sync_copy(rows, acc_sh.at[dstv.at[j]], add=True)
            return carry
        lax.fori_loop(0, R_PER_W // IDX_BLK, blk_body, 0)
        plsc.subcore_barrier()

        def out_body(k, carry):
            r0 = s * ROWS_PER_TILE + k * CHUNK
            pltpu.sync_copy(acc_sh.at[pl.ds(r0, CHUNK)], rows)
            pltpu.sync_copy(rows, out_hbm.at[pl.ds(c * N_PAD + r0, CHUNK)])
            return carry
        lax.fori_loop(0, ROWS_PER_TILE // CHUNK, out_body, 0)

    def call(*args):
        mesh = plsc.VectorSubcoreMesh(
            core_axis_name="c", subcore_axis_name="s",
            num_cores=NC, num_subcores=NS,
        )
        return pl.kernel(
            body, out_type=out_type, mesh=mesh, scratch_types=scratch
        )(*args)

    return call


_seg_sum = _make_seg_sum(D_IN, gather=True)
_seg_deg = _make_seg_sum(D_IN, gather=False)

ROW_BLK = 2000
GRID = N // ROW_BLK


def _tc1_body(x_ref, agg_ref, deg_ref, w1t_ref, b1_ref, h_ref):
    agg = agg_ref[0] + agg_ref[1] + x_ref[...]
    deg = deg_ref[0, :, 0:1] + deg_ref[1, :, 0:1]
    hn = agg * (1.0 / (deg + 1.0))
    h = jnp.dot(hn, w1t_ref[...], preferred_element_type=jnp.float32)
    h = h + b1_ref[...]
    h_ref[...] = jnp.where(h >= 0.0, h, 0.01 * h)


def _tc2_body(h_ref, agg_ref, deg_ref, w2t_ref, b2_ref, o_ref):
    agg = agg_ref[0] + agg_ref[1] + h_ref[...]
    deg = deg_ref[0, :, 0:1] + deg_ref[1, :, 0:1]
    hn = agg * (1.0 / (deg + 1.0))
    o = jnp.dot(hn, w2t_ref[...], preferred_element_type=jnp.float32)
    o_ref[...] = o + b2_ref[...]


_tc1 = pl.pallas_call(
    _tc1_body,
    grid=(GRID,),
    in_specs=[
        pl.BlockSpec((ROW_BLK, D_IN), lambda i: (i, 0)),
        pl.BlockSpec((NC, ROW_BLK, D_IN), lambda i: (0, i, 0)),
        pl.BlockSpec((NC, ROW_BLK, D_IN), lambda i: (0, i, 0)),
        pl.BlockSpec((D_IN, D_H), lambda i: (0, 0)),
        pl.BlockSpec((1, D_H), lambda i: (0, 0)),
    ],
    out_specs=pl.BlockSpec((ROW_BLK, D_H), lambda i: (i, 0)),
    out_shape=jax.ShapeDtypeStruct((N, D_H), jnp.float32),
)

_tc2 = pl.pallas_call(
    _tc2_body,
    grid=(GRID,),
    in_specs=[
        pl.BlockSpec((ROW_BLK, D_H), lambda i: (i, 0)),
        pl.BlockSpec((NC, ROW_BLK, D_H), lambda i: (0, i, 0)),
        pl.BlockSpec((NC, ROW_BLK, D_IN), lambda i: (0, i, 0)),
        pl.BlockSpec((D_H, D_OUT), lambda i: (0, 0)),
        pl.BlockSpec((1, D_OUT), lambda i: (0, 0)),
    ],
    out_specs=pl.BlockSpec((ROW_BLK, D_OUT), lambda i: (i, 0)),
    out_shape=jax.ShapeDtypeStruct((N, D_OUT), jnp.float32),
)


def kernel(in_feat, edge_index, W1, b1, W2, b2):
    src = edge_index[0]
    dst = edge_index[1]
    pad = E_PAD - E
    srcp = jnp.concatenate([src, jnp.zeros((pad,), jnp.int32)])
    srcp = srcp.reshape(IDX_ROWS_PAD, CHUNK)
    dstp = jnp.concatenate([dst, jnp.full((pad,), N, jnp.int32)])
    dstp = dstp.reshape(IDX_ROWS_PAD, CHUNK)

    zrows = jnp.zeros((CHUNK, D_IN), jnp.float32)
    ones = jnp.ones((CHUNK, D_IN), jnp.float32)

    (degf,) = _seg_deg(ones, srcp, dstp, zrows)
    degp = degf.reshape(NC, N_PAD, D_IN)
    (agg1f,) = _seg_sum(in_feat, srcp, dstp, zrows)
    agg1p = agg1f.reshape(NC, N_PAD, D_IN)
    h = _tc1(in_feat, agg1p, degp, W1.T, b1.reshape(1, D_H))
    (agg2f,) = _seg_sum(h, srcp, dstp, zrows)
    agg2p = agg2f.reshape(NC, N_PAD, D_H)
    return _tc2(h, agg2p, degp, W2.T, b2.reshape(1, D_OUT))


# speedup vs baseline: 3.1353x; 1.0703x over previous
import jax
import jax.numpy as jnp
from jax import lax
from jax.experimental import pallas as pl
from jax.experimental.pallas import tpu as pltpu
from jax.experimental.pallas import tpu_sc as plsc

N = 10000
E = 320000
D_IN = 128
D_H = 128
D_OUT = 64

NC = 2
NS = 16
NW = NC * NS
CHUNK = 128
IDX_ROWS = (E + CHUNK - 1) // CHUNK
IDX_ROWS_PAD = ((IDX_ROWS + 8 * NW - 1) // (8 * NW)) * (8 * NW)
E_PAD = IDX_ROWS_PAD * CHUNK
R_PER_W = IDX_ROWS_PAD // NW
IDX_BLK = 16
N_PAD = 10240
ROWS_PER_TILE = N_PAD // NS


def _make_seg_sum(d, gather):
    out_type = [jax.ShapeDtypeStruct((NC * N_PAD, d), jnp.float32)]

    scratch = [
        pltpu.VMEM((IDX_BLK, CHUNK), jnp.int32),
        pltpu.VMEM((IDX_BLK, CHUNK), jnp.int32),
        pltpu.VMEM((CHUNK, d), jnp.float32),
        pltpu.VMEM((CHUNK, d), jnp.float32),
        pltpu.VMEM_SHARED((N_PAD, d), jnp.float32),
        pltpu.SemaphoreType.DMA,
        pltpu.SemaphoreType.DMA,
    ]

    def body(x_hbm, src_hbm, dst_hbm, z_hbm, out_hbm,
             srcv, dstv, rows, rows1, acc_sh, sem, ssem):
        c = lax.axis_index("c")
        s = lax.axis_index("s")
        w = s * NC + c

        pltpu.sync_copy(z_hbm, rows)

        def zero_body(k, carry):
            r0 = s * ROWS_PER_TILE + k * CHUNK
            pltpu.sync_copy(rows, acc_sh.at[pl.ds(r0, CHUNK)])
            return carry
        lax.fori_loop(0, ROWS_PER_TILE // CHUNK, zero_body, 0)
        plsc.subcore_barrier()

        if not gather:
            pltpu.sync_copy(x_hbm, rows)

        bufs = (rows, rows1)

        def blk_body(b, carry):
            r0 = w * R_PER_W + b * IDX_BLK
            if gather:
                pltpu.sync_copy(src_hbm.at[pl.ds(r0, IDX_BLK)], srcv)
            pltpu.sync_copy(dst_hbm.at[pl.ds(r0, IDX_BLK)], dstv)

            if gather:
                scat = []
                for j in range(IDX_BLK):
                    buf = bufs[j % 2]
                    if j >= 2:
                        scat[j - 2].wait()
                    pltpu.async_copy(x_hbm.at[srcv.at[j]], buf, sem).wait()
                    scat.append(pltpu.async_copy(
                        buf, acc_sh.at[dstv.at[j]], ssem, add=True))
                scat[IDX_BLK - 2].wait()
                scat[IDX_BLK - 1].wait()
            else:
                for j in range(IDX_BLK):
                    pltpu.sync_copy(rows, acc_sh.at[dstv.at[j]], add=True)
            return carry
        lax.fori_loop(0, R_PER_W // IDX_BLK, blk_body, 0)
        plsc.subcore_barrier()

        def out_body(k, carry):
            r0 = s * ROWS_PER_TILE + k * CHUNK
            pltpu.sync_copy(acc_sh.at[pl.ds(r0, CHUNK)], rows)
            pltpu.sync_copy(rows, out_hbm.at[pl.ds(c * N_PAD + r0, CHUNK)])
            return carry
        lax.fori_loop(0, ROWS_PER_TILE // CHUNK, out_body, 0)

    def call(*args):
        mesh = plsc.VectorSubcoreMesh(
            core_axis_name="c", subcore_axis_name="s",
            num_cores=NC, num_subcores=NS,
        )
        return pl.kernel(
            body, out_type=out_type, mesh=mesh, scratch_types=scratch
        )(*args)

    return call


_seg_sum = _make_seg_sum(D_IN, gather=True)
_seg_deg = _make_seg_sum(D_IN, gather=False)

ROW_BLK = 2000
GRID = N // ROW_BLK


def _tc1_body(x_ref, agg_ref, deg_ref, w1t_ref, b1_ref, h_ref):
    agg = agg_ref[0] + agg_ref[1] + x_ref[...]
    deg = deg_ref[0, :, 0:1] + deg_ref[1, :, 0:1]
    hn = agg * (1.0 / (deg + 1.0))
    h = jnp.dot(hn, w1t_ref[...], preferred_element_type=jnp.float32)
    h = h + b1_ref[...]
    h_ref[...] = jnp.where(h >= 0.0, h, 0.01 * h)


def _tc2_body(h_ref, agg_ref, deg_ref, w2t_ref, b2_ref, o_ref):
    agg = agg_ref[0] + agg_ref[1] + h_ref[...]
    deg = deg_ref[0, :, 0:1] + deg_ref[1, :, 0:1]
    hn = agg * (1.0 / (deg + 1.0))
    o = jnp.dot(hn, w2t_ref[...], preferred_element_type=jnp.float32)
    o_ref[...] = o + b2_ref[...]


_tc1 = pl.pallas_call(
    _tc1_body,
    grid=(GRID,),
    in_specs=[
        pl.BlockSpec((ROW_BLK, D_IN), lambda i: (i, 0)),
        pl.BlockSpec((NC, ROW_BLK, D_IN), lambda i: (0, i, 0)),
        pl.BlockSpec((NC, ROW_BLK, D_IN), lambda i: (0, i, 0)),
        pl.BlockSpec((D_IN, D_H), lambda i: (0, 0)),
        pl.BlockSpec((1, D_H), lambda i: (0, 0)),
    ],
    out_specs=pl.BlockSpec((ROW_BLK, D_H), lambda i: (i, 0)),
    out_shape=jax.ShapeDtypeStruct((N, D_H), jnp.float32),
)

_tc2 = pl.pallas_call(
    _tc2_body,
    grid=(GRID,),
    in_specs=[
        pl.BlockSpec((ROW_BLK, D_H), lambda i: (i, 0)),
        pl.BlockSpec((NC, ROW_BLK, D_H), lambda i: (0, i, 0)),
        pl.BlockSpec((NC, ROW_BLK, D_IN), lambda i: (0, i, 0)),
        pl.BlockSpec((D_H, D_OUT), lambda i: (0, 0)),
        pl.BlockSpec((1, D_OUT), lambda i: (0, 0)),
    ],
    out_specs=pl.BlockSpec((ROW_BLK, D_OUT), lambda i: (i, 0)),
    out_shape=jax.ShapeDtypeStruct((N, D_OUT), jnp.float32),
)


def kernel(in_feat, edge_index, W1, b1, W2, b2):
    src = edge_index[0]
    dst = edge_index[1]
    pad = E_PAD - E
    srcp = jnp.concatenate([src, jnp.zeros((pad,), jnp.int32)])
    srcp = srcp.reshape(IDX_ROWS_PAD, CHUNK)
    dstp = jnp.concatenate([dst, jnp.full((pad,), N, jnp.int32)])
    dstp = dstp.reshape(IDX_ROWS_PAD, CHUNK)

    zrows = jnp.zeros((CHUNK, D_IN), jnp.float32)
    ones = jnp.ones((CHUNK, D_IN), jnp.float32)

    (degf,) = _seg_deg(ones, srcp, dstp, zrows)
    degp = degf.reshape(NC, N_PAD, D_IN)
    (agg1f,) = _seg_sum(in_feat, srcp, dstp, zrows)
    agg1p = agg1f.reshape(NC, N_PAD, D_IN)
    h = _tc1(in_feat, agg1p, degp, W1.T, b1.reshape(1, D_H))
    (agg2f,) = _seg_sum(h, srcp, dstp, zrows)
    agg2p = agg2f.reshape(NC, N_PAD, D_H)
    return _tc2(h, agg2p, degp, W2.T, b2.reshape(1, D_OUT))


# speedup vs baseline: 4.2798x; 1.3650x over previous
import jax
import jax.numpy as jnp
from jax import lax
from jax.experimental import pallas as pl
from jax.experimental.pallas import tpu as pltpu
from jax.experimental.pallas import tpu_sc as plsc

N = 10000
E = 320000
D_IN = 128
D_H = 128
D_OUT = 64

NC = 2
NS = 16
NW = NC * NS
CHUNK = 128
IDX_ROWS = (E + CHUNK - 1) // CHUNK
IDX_ROWS_PAD = ((IDX_ROWS + 8 * NW - 1) // (8 * NW)) * (8 * NW)
E_PAD = IDX_ROWS_PAD * CHUNK
R_PER_W = IDX_ROWS_PAD // NW
IDX_BLK = 16
N_PAD = 10240
ROWS_PER_TILE = N_PAD // NS


def _make_seg_sum(d, gather, untiled=False):
    out_type = [jax.ShapeDtypeStruct((NC * N_PAD, d), jnp.float32)]

    scratch = [
        pltpu.VMEM((IDX_BLK, CHUNK), jnp.int32),
        pltpu.VMEM((IDX_BLK, CHUNK), jnp.int32),
        pltpu.VMEM((CHUNK, d), jnp.float32),
        pltpu.VMEM((CHUNK, d), jnp.float32),
        pltpu.VMEM_SHARED((N_PAD, d), jnp.float32),
        pltpu.SemaphoreType.DMA,
        pltpu.SemaphoreType.DMA,
    ]

    def body(x_hbm, src_hbm, dst_hbm, z_hbm, out_hbm,
             srcv, dstv, rows, rows1, acc_sh, sem, ssem):
        c = lax.axis_index("c")
        s = lax.axis_index("s")
        w = s * NC + c

        pltpu.sync_copy(z_hbm, rows)

        def zero_body(k, carry):
            r0 = s * ROWS_PER_TILE + k * CHUNK
            pltpu.sync_copy(rows, acc_sh.at[pl.ds(r0, CHUNK)])
            return carry
        lax.fori_loop(0, ROWS_PER_TILE // CHUNK, zero_body, 0)
        plsc.subcore_barrier()

        if not gather:
            pltpu.sync_copy(x_hbm, rows)

        bufs = (rows, rows1)

        def blk_body(b, carry):
            r0 = w * R_PER_W + b * IDX_BLK
            if gather:
                pltpu.sync_copy(src_hbm.at[pl.ds(r0, IDX_BLK)], srcv)
            pltpu.sync_copy(dst_hbm.at[pl.ds(r0, IDX_BLK)], dstv)

            if gather:
                scat = []
                for j in range(IDX_BLK):
                    buf = bufs[j % 2]
                    if j >= 2:
                        scat[j - 2].wait()
                    pltpu.async_copy(x_hbm.at[srcv.at[j]], buf, sem).wait()
                    scat.append(pltpu.async_copy(
                        buf, acc_sh.at[dstv.at[j]], ssem, add=True))
                scat[IDX_BLK - 2].wait()
                scat[IDX_BLK - 1].wait()
            else:
                for j in range(IDX_BLK):
                    pltpu.sync_copy(rows, acc_sh.at[dstv.at[j]], add=True)
            return carry
        lax.fori_loop(0, R_PER_W // IDX_BLK, blk_body, 0)
        plsc.subcore_barrier()

        def out_body(k, carry):
            r0 = s * ROWS_PER_TILE + k * CHUNK
            pltpu.sync_copy(acc_sh.at[pl.ds(r0, CHUNK)], rows)
            pltpu.sync_copy(rows, out_hbm.at[pl.ds(c * N_PAD + r0, CHUNK)])
            return carry
        lax.fori_loop(0, ROWS_PER_TILE // CHUNK, out_body, 0)

    def call(*args):
        mesh = plsc.VectorSubcoreMesh(
            core_axis_name="c", subcore_axis_name="s",
            num_cores=NC, num_subcores=NS,
        )
        params = (pltpu.CompilerParams(use_tc_tiling_on_sc=False)
                  if untiled else None)
        return pl.kernel(
            body, out_type=out_type, mesh=mesh, scratch_types=scratch,
            compiler_params=params,
        )(*args)

    return call


_seg_sum = _make_seg_sum(D_IN, gather=True)
_seg_sum_64 = _make_seg_sum(D_OUT, gather=True, untiled=True)
_seg_deg = _make_seg_sum(D_IN, gather=False)

ROW_BLK = 2000
GRID = N // ROW_BLK


def _tc1_body(x_ref, agg_ref, deg_ref, w1t_ref, b1_ref, w2t_ref, p_ref):
    agg = agg_ref[0] + agg_ref[1] + x_ref[...]
    deg = deg_ref[0, :, 0:1] + deg_ref[1, :, 0:1]
    hn = agg * (1.0 / (deg + 1.0))
    h = jnp.dot(hn, w1t_ref[...], preferred_element_type=jnp.float32)
    h = h + b1_ref[...]
    h = jnp.where(h >= 0.0, h, 0.01 * h)
    p_ref[...] = jnp.dot(h, w2t_ref[...], preferred_element_type=jnp.float32)


def _tc2_body(p_ref, agg_ref, deg_ref, b2_ref, o_ref):
    agg = agg_ref[0] + agg_ref[1] + p_ref[...]
    deg = deg_ref[0, :, 0:1] + deg_ref[1, :, 0:1]
    o_ref[...] = agg * (1.0 / (deg + 1.0)) + b2_ref[...]


_tc1 = pl.pallas_call(
    _tc1_body,
    grid=(GRID,),
    in_specs=[
        pl.BlockSpec((ROW_BLK, D_IN), lambda i: (i, 0)),
        pl.BlockSpec((NC, ROW_BLK, D_IN), lambda i: (0, i, 0)),
        pl.BlockSpec((NC, ROW_BLK, D_IN), lambda i: (0, i, 0)),
        pl.BlockSpec((D_IN, D_H), lambda i: (0, 0)),
        pl.BlockSpec((1, D_H), lambda i: (0, 0)),
        pl.BlockSpec((D_H, D_OUT), lambda i: (0, 0)),
    ],
    out_specs=pl.BlockSpec((ROW_BLK, D_OUT), lambda i: (i, 0)),
    out_shape=jax.ShapeDtypeStruct((N, D_OUT), jnp.float32),
)

_tc2 = pl.pallas_call(
    _tc2_body,
    grid=(GRID,),
    in_specs=[
        pl.BlockSpec((ROW_BLK, D_OUT), lambda i: (i, 0)),
        pl.BlockSpec((NC, ROW_BLK, D_OUT), lambda i: (0, i, 0)),
        pl.BlockSpec((NC, ROW_BLK, D_IN), lambda i: (0, i, 0)),
        pl.BlockSpec((1, D_OUT), lambda i: (0, 0)),
    ],
    out_specs=pl.BlockSpec((ROW_BLK, D_OUT), lambda i: (i, 0)),
    out_shape=jax.ShapeDtypeStruct((N, D_OUT), jnp.float32),
)


def kernel(in_feat, edge_index, W1, b1, W2, b2):
    src = edge_index[0]
    dst = edge_index[1]
    pad = E_PAD - E
    srcp = jnp.concatenate([src, jnp.zeros((pad,), jnp.int32)])
    srcp = srcp.reshape(IDX_ROWS_PAD, CHUNK)
    dstp = jnp.concatenate([dst, jnp.full((pad,), N, jnp.int32)])
    dstp = dstp.reshape(IDX_ROWS_PAD, CHUNK)

    zrows = jnp.zeros((CHUNK, D_IN), jnp.float32)
    ones = jnp.ones((CHUNK, D_IN), jnp.float32)

    (degf,) = _seg_deg(ones, srcp, dstp, zrows)
    degp = degf.reshape(NC, N_PAD, D_IN)
    (agg1f,) = _seg_sum(in_feat, srcp, dstp, zrows)
    agg1p = agg1f.reshape(NC, N_PAD, D_IN)
    p = _tc1(in_feat, agg1p, degp, W1.T, b1.reshape(1, D_H), W2.T)
    (agg2f,) = _seg_sum_64(p, srcp, dstp, zrows[:, :D_OUT])
    agg2p = agg2f.reshape(NC, N_PAD, D_OUT)
    return _tc2(p, agg2p, degp, b2.reshape(1, D_OUT))


# speedup vs baseline: 4.3806x; 1.0236x over previous
import jax
import jax.numpy as jnp
from jax import lax
from jax.experimental import pallas as pl
from jax.experimental.pallas import tpu as pltpu
from jax.experimental.pallas import tpu_sc as plsc

N = 10000
E = 320000
D_IN = 128
D_H = 128
D_OUT = 64

NC = 2
NS = 16
NW = NC * NS
CHUNK = 128
IDX_ROWS = (E + CHUNK - 1) // CHUNK
IDX_ROWS_PAD = ((IDX_ROWS + 8 * NW - 1) // (8 * NW)) * (8 * NW)
E_PAD = IDX_ROWS_PAD * CHUNK
R_PER_W = IDX_ROWS_PAD // NW
IDX_BLK = 16
N_PAD = 10240
ROWS_PER_TILE = N_PAD // NS


def _make_seg_sum(d, gather, untiled=False):
    out_type = [jax.ShapeDtypeStruct((NC * N_PAD, d), jnp.float32)]

    scratch = [
        pltpu.VMEM((IDX_BLK, CHUNK), jnp.int32),
        pltpu.VMEM((IDX_BLK, CHUNK), jnp.int32),
        pltpu.VMEM((CHUNK, d), jnp.float32),
        pltpu.VMEM((CHUNK, d), jnp.float32),
        pltpu.VMEM_SHARED((N_PAD, d), jnp.float32),
        pltpu.SemaphoreType.DMA,
        pltpu.SemaphoreType.DMA,
    ]

    def body(x_hbm, src_hbm, dst_hbm, z_hbm, out_hbm,
             srcv, dstv, rows, rows1, acc_sh, sem, ssem):
        c = lax.axis_index("c")
        s = lax.axis_index("s")
        w = s * NC + c

        pltpu.sync_copy(z_hbm, rows)

        def zero_body(k, carry):
            r0 = s * ROWS_PER_TILE + k * CHUNK
            pltpu.sync_copy(rows, acc_sh.at[pl.ds(r0, CHUNK)])
            return carry
        lax.fori_loop(0, ROWS_PER_TILE // CHUNK, zero_body, 0)
        plsc.subcore_barrier()

        if not gather:
            pltpu.sync_copy(x_hbm, rows)

        bufs = (rows, rows1)

        def blk_body(b, carry):
            r0 = w * R_PER_W + b * IDX_BLK
            if gather:
                pltpu.sync_copy(src_hbm.at[pl.ds(r0, IDX_BLK)], srcv)
            pltpu.sync_copy(dst_hbm.at[pl.ds(r0, IDX_BLK)], dstv)

            if gather:
                scat = []
                for j in range(IDX_BLK):
                    buf = bufs[j % 2]
                    if j >= 2:
                        scat[j - 2].wait()
                    pltpu.async_copy(x_hbm.at[srcv.at[j]], buf, sem).wait()
                    scat.append(pltpu.async_copy(
                        buf, acc_sh.at[dstv.at[j]], ssem, add=True))
                scat[IDX_BLK - 2].wait()
                scat[IDX_BLK - 1].wait()
            else:
                for j in range(IDX_BLK):
                    pltpu.sync_copy(rows, acc_sh.at[dstv.at[j]], add=True)
            return carry
        lax.fori_loop(0, R_PER_W // IDX_BLK, blk_body, 0)
        plsc.subcore_barrier()

        def out_body(k, carry):
            r0 = s * ROWS_PER_TILE + k * CHUNK
            pltpu.sync_copy(acc_sh.at[pl.ds(r0, CHUNK)], rows)
            pltpu.sync_copy(rows, out_hbm.at[pl.ds(c * N_PAD + r0, CHUNK)])
            return carry
        lax.fori_loop(0, ROWS_PER_TILE // CHUNK, out_body, 0)

    def call(*args):
        mesh = plsc.VectorSubcoreMesh(
            core_axis_name="c", subcore_axis_name="s",
            num_cores=NC, num_subcores=NS,
        )
        params = (pltpu.CompilerParams(use_tc_tiling_on_sc=False)
                  if untiled else None)
        return pl.kernel(
            body, out_type=out_type, mesh=mesh, scratch_types=scratch,
            compiler_params=params,
        )(*args)

    return call


_seg_sum = _make_seg_sum(D_IN, gather=True, untiled=True)
_seg_sum_64 = _make_seg_sum(D_OUT, gather=True, untiled=True)
_seg_deg = _make_seg_sum(D_OUT, gather=False, untiled=True)

ROW_BLK = 2000
GRID = N // ROW_BLK


def _tc1_body(x_ref, agg_ref, deg_ref, w1t_ref, b1_ref, w2t_ref, p_ref):
    agg = agg_ref[0] + agg_ref[1] + x_ref[...]
    deg = deg_ref[0, :, 0:1] + deg_ref[1, :, 0:1]
    hn = agg * (1.0 / (deg + 1.0))
    h = jnp.dot(hn, w1t_ref[...], preferred_element_type=jnp.float32)
    h = h + b1_ref[...]
    h = jnp.where(h >= 0.0, h, 0.01 * h)
    p_ref[...] = jnp.dot(h, w2t_ref[...], preferred_element_type=jnp.float32)


def _tc2_body(p_ref, agg_ref, deg_ref, b2_ref, o_ref):
    agg = agg_ref[0] + agg_ref[1] + p_ref[...]
    deg = deg_ref[0, :, 0:1] + deg_ref[1, :, 0:1]
    o_ref[...] = agg * (1.0 / (deg + 1.0)) + b2_ref[...]


_tc1 = pl.pallas_call(
    _tc1_body,
    grid=(GRID,),
    in_specs=[
        pl.BlockSpec((ROW_BLK, D_IN), lambda i: (i, 0)),
        pl.BlockSpec((NC, ROW_BLK, D_IN), lambda i: (0, i, 0)),
        pl.BlockSpec((NC, ROW_BLK, D_OUT), lambda i: (0, i, 0)),
        pl.BlockSpec((D_IN, D_H), lambda i: (0, 0)),
        pl.BlockSpec((1, D_H), lambda i: (0, 0)),
        pl.BlockSpec((D_H, D_OUT), lambda i: (0, 0)),
    ],
    out_specs=pl.BlockSpec((ROW_BLK, D_OUT), lambda i: (i, 0)),
    out_shape=jax.ShapeDtypeStruct((N, D_OUT), jnp.float32),
)

_tc2 = pl.pallas_call(
    _tc2_body,
    grid=(GRID,),
    in_specs=[
        pl.BlockSpec((ROW_BLK, D_OUT), lambda i: (i, 0)),
        pl.BlockSpec((NC, ROW_BLK, D_OUT), lambda i: (0, i, 0)),
        pl.BlockSpec((NC, ROW_BLK, D_OUT), lambda i: (0, i, 0)),
        pl.BlockSpec((1, D_OUT), lambda i: (0, 0)),
    ],
    out_specs=pl.BlockSpec((ROW_BLK, D_OUT), lambda i: (i, 0)),
    out_shape=jax.ShapeDtypeStruct((N, D_OUT), jnp.float32),
)


def kernel(in_feat, edge_index, W1, b1, W2, b2):
    src = edge_index[0]
    dst = edge_index[1]
    pad = E_PAD - E
    srcp = jnp.concatenate([src, jnp.zeros((pad,), jnp.int32)])
    srcp = srcp.reshape(IDX_ROWS_PAD, CHUNK)
    dstp = jnp.concatenate([dst, jnp.full((pad,), N, jnp.int32)])
    dstp = dstp.reshape(IDX_ROWS_PAD, CHUNK)

    zrows = jnp.zeros((CHUNK, D_IN), jnp.float32)
    ones = jnp.ones((CHUNK, D_IN), jnp.float32)

    (degf,) = _seg_deg(ones[:, :D_OUT], srcp, dstp, zrows[:, :D_OUT])
    degp = degf.reshape(NC, N_PAD, D_OUT)
    (agg1f,) = _seg_sum(in_feat, srcp, dstp, zrows)
    agg1p = agg1f.reshape(NC, N_PAD, D_IN)
    p = _tc1(in_feat, agg1p, degp, W1.T, b1.reshape(1, D_H), W2.T)
    (agg2f,) = _seg_sum_64(p, srcp, dstp, zrows[:, :D_OUT])
    agg2p = agg2f.reshape(NC, N_PAD, D_OUT)
    return _tc2(p, agg2p, degp, b2.reshape(1, D_OUT))
